# Initial kernel scaffold; baseline (speedup 1.0000x reference)
import jax
import jax.numpy as jnp
from jax.experimental import pallas as pl


def kernel(X, vertex, edges, W, eps):
    raise NotImplementedError("write your pallas kernel here")


# speedup vs baseline: 4.8979x; 4.8979x over previous
import functools

import jax
import jax.numpy as jnp
from jax import lax
from jax.experimental import pallas as pl
from jax.experimental.pallas import tpu as pltpu
from jax.experimental.pallas import tpu_sc as plsc

NC = 2
NS = 16
L = 16
NW = NC * NS

N = 10000
E = 10000
D = 128
R_PAD = 10240
TRASH = 10000
CHUNK = 128
K = 79
NNZ_PAD = NW * K * CHUNK
RPT = R_PAD // NS
RPW = R_PAD // NW
NB = 64


def _mesh():
  return plsc.VectorSubcoreMesh(
      core_axis_name="c", subcore_axis_name="s", num_cores=NC, num_subcores=NS
  )


def _scatter_body(with_counts, *refs):
  if with_counts:
    (src_hbm, gidx_hbm, sidx_hbm, out_sums, out_cnt,
     gidx_v, sidx_v, rows_v, cnt_v, acc_sh, sem) = refs
  else:
    (src_hbm, gidx_hbm, sidx_hbm, out_sums,
     gidx_v, sidx_v, rows_v, acc_sh, sem) = refs

  c = lax.axis_index("c")
  s = lax.axis_index("s")
  wid = c * NS + s
  base = s * RPT

  zv = jnp.zeros((L,), jnp.float32)

  def zero_row(i, carry):
    for j in range(D // L):
      rows_v[i, pl.ds(j * L, L)] = zv
    return carry

  lax.fori_loop(0, CHUNK, zero_row, 0)

  if with_counts:
    def zero_cnt(i, carry):
      cnt_v[pl.ds(i * L, L)] = zv
      return carry

    lax.fori_loop(0, R_PAD // L, zero_cnt, 0)

  for k in range(RPT // CHUNK):
    pltpu.sync_copy(rows_v, acc_sh.at[pl.ds(base + k * CHUNK, CHUNK)])

  pltpu.sync_copy(gidx_hbm.at[wid], gidx_v)
  pltpu.sync_copy(sidx_hbm.at[wid], sidx_v)

  plsc.subcore_barrier()

  ones = jnp.ones((L,), jnp.float32)

  def step(j, carry):
    pltpu.async_copy(src_hbm.at[gidx_v.at[j]], rows_v, sem).wait()
    pltpu.sync_copy(rows_v, acc_sh.at[sidx_v.at[j]], add=True)
    if with_counts:
      for t in range(CHUNK // L):
        idx = sidx_v[j, pl.ds(t * L, L)]
        plsc.addupdate_scatter(cnt_v, [idx], ones)
    return carry

  lax.fori_loop(0, K, step, 0)

  if with_counts:
    pltpu.sync_copy(cnt_v, out_cnt.at[c, s])

  plsc.subcore_barrier()

  for k in range(RPT // CHUNK):
    r0 = base + k * CHUNK
    pltpu.sync_copy(acc_sh.at[pl.ds(r0, CHUNK)], rows_v)
    pltpu.sync_copy(rows_v, out_sums.at[c, pl.ds(r0, CHUNK)])


def _make_scatter(with_counts):
  outs = [jax.ShapeDtypeStruct((NC, R_PAD, D), jnp.float32)]
  scratch = [
      pltpu.VMEM((K, CHUNK), jnp.int32),
      pltpu.VMEM((K, CHUNK), jnp.int32),
      pltpu.VMEM((CHUNK, D), jnp.float32),
  ]
  if with_counts:
    outs.append(jax.ShapeDtypeStruct((NC, NS, R_PAD), jnp.float32))
    scratch.append(pltpu.VMEM((R_PAD,), jnp.float32))
  scratch.append(pltpu.VMEM_SHARED((R_PAD, D), jnp.float32))
  scratch.append(pltpu.SemaphoreType.DMA)
  return pl.kernel(
      functools.partial(_scatter_body, with_counts),
      out_type=tuple(outs) if with_counts else outs[0],
      mesh=_mesh(),
      scratch_types=scratch,
      compiler_params=pltpu.CompilerParams(needs_layout_passes=False),
  )


def _norm_body(sums_hbm, cnt_hbm, out_hbm, s0, s1, cbuf, inv_v, o):
  c = lax.axis_index("c")
  s = lax.axis_index("s")
  wid = c * NS + s
  base = wid * RPW

  for k in range(RPW // NB):
    r0 = base + k * NB
    pltpu.sync_copy(sums_hbm.at[0, pl.ds(r0, NB)], s0)
    pltpu.sync_copy(sums_hbm.at[1, pl.ds(r0, NB)], s1)
    for w2 in range(NW):
      pltpu.sync_copy(cnt_hbm.at[w2, pl.ds(r0, NB)], cbuf.at[w2])

    for g in range(NB // L):
      tot = cbuf[0, pl.ds(g * L, L)]
      for w2 in range(1, NW):
        tot = tot + cbuf[w2, pl.ds(g * L, L)]
      inv_v[pl.ds(g * L, L)] = 1.0 / jnp.maximum(tot, 1.0)

    def row(i, carry):
      iv = plsc.load_gather(inv_v, [jnp.full((L,), i, jnp.int32)])
      for j in range(D // L):
        o[i, pl.ds(j * L, L)] = (
            s0[i, pl.ds(j * L, L)] + s1[i, pl.ds(j * L, L)]
        ) * iv
      return carry

    lax.fori_loop(0, NB, row, 0)
    pltpu.sync_copy(o, out_hbm.at[pl.ds(r0, NB)])


def _make_norm():
  return pl.kernel(
      _norm_body,
      out_type=jax.ShapeDtypeStruct((R_PAD, D), jnp.float32),
      mesh=_mesh(),
      scratch_types=[
          pltpu.VMEM((NB, D), jnp.float32),
          pltpu.VMEM((NB, D), jnp.float32),
          pltpu.VMEM((NW, NB), jnp.float32),
          pltpu.VMEM((NB,), jnp.float32),
          pltpu.VMEM((NB, D), jnp.float32),
      ],
      compiler_params=pltpu.CompilerParams(needs_layout_passes=False),
  )


def _mm_body(eps_ref, x_ref, v0_ref, v1_ref, w_ref, o_ref):
  scale = 1.0 + eps_ref[0]
  acc = scale * x_ref[...] + v0_ref[...] + v1_ref[...]
  o_ref[...] = lax.dot_general(
      acc, w_ref[...], (((1,), (1,)), ((), ())),
      preferred_element_type=jnp.float32,
  )


def _matmul(eps, Xp, v0, v1, W):
  M = Xp.shape[0]
  BM = 1280
  return pl.pallas_call(
      _mm_body,
      grid=(M // BM,),
      in_specs=[
          pl.BlockSpec(memory_space=pltpu.SMEM),
          pl.BlockSpec((BM, D), lambda i: (i, 0)),
          pl.BlockSpec((BM, D), lambda i: (i, 0)),
          pl.BlockSpec((BM, D), lambda i: (i, 0)),
          pl.BlockSpec((D, D), lambda i: (0, 0)),
      ],
      out_specs=pl.BlockSpec((BM, D), lambda i: (i, 0)),
      out_shape=jax.ShapeDtypeStruct((M, D), jnp.float32),
  )(eps, Xp, v0, v1, W)


def kernel(X, vertex, edges, W, eps):
  Xp = jnp.pad(X, ((0, R_PAD - N), (0, 0)))
  npad = NNZ_PAD - vertex.shape[0]
  fill = jnp.full((npad,), TRASH, jnp.int32)
  v3 = jnp.concatenate([vertex.astype(jnp.int32), fill]).reshape(NW, K, CHUNK)
  e3 = jnp.concatenate([edges.astype(jnp.int32), fill]).reshape(NW, K, CHUNK)

  sums, cnts = _make_scatter(True)(Xp, v3, e3)
  Xe = _make_norm()(sums, cnts.reshape(NW, R_PAD))
  xv = _make_scatter(False)(Xe, e3, v3)
  out = _matmul(eps, Xp, xv[0], xv[1], W)
  return out[:N]
